# baseline (speedup 1.0000x reference)
import functools

import jax
import jax.numpy as jnp
import numpy as np
from jax import lax
from jax.experimental import pallas as pl
from jax.experimental.pallas import tpu as pltpu
from jax.experimental.pallas import tpu_sc as plsc

F32 = jnp.float32


def _sc_gather_multi(tables, idxs, CH=40, NB=5):
    E = idxs[0].shape[0]
    ns = len(tables)
    Ws = [t.shape[1] for t in tables]
    dts = [t.dtype for t in tables]
    Nt = tables[0].shape[0]
    assert all(t.shape == (Nt, Ws[0]) and t.dtype == dts[0] for t in tables)
    per_w = E // 32
    n_ch = per_w // CH
    assert per_w % CH == 0 and n_ch % NB == 0 and CH % 8 == 0
    n_outer = n_ch // NB
    CZ = 80
    n_zc = Nt // CZ
    n_j = -(-n_zc // 16)
    assert Nt % CZ == 0
    mesh = plsc.VectorSubcoreMesh(core_axis_name="c", subcore_axis_name="s")

    @functools.partial(
        pl.kernel,
        out_type=tuple(
            jax.ShapeDtypeStruct((E, Ws[i]), dts[i]) for i in range(ns)),
        mesh=mesh,
        scratch_types=[pltpu.VMEM((per_w,), jnp.int32),
                       pltpu.VMEM_SHARED((Nt, Ws[0]), dts[0])],
    )
    def k(*refs):
        tabs = refs[:ns]
        ihs = refs[ns:2 * ns]
        outs = refs[2 * ns:3 * ns]
        idx_v = refs[3 * ns]
        stage = refs[3 * ns + 1]
        c = lax.axis_index("c")
        s = lax.axis_index("s")
        base = (s * 2 + c) * per_w

        def load_stage(tab):
            for j in range(n_j):
                cid = s + 16 * j

                @pl.when(cid < n_zc)
                def _():
                    pltpu.sync_copy(tab.at[pl.ds(cid * CZ, CZ)],
                                    stage.at[pl.ds(cid * CZ, CZ)])
            plsc.subcore_barrier()

        def run(tab, ih, out, rows, gsem, wsem):
            pltpu.sync_copy(ih.at[pl.ds(base, per_w)], idx_v)

            def gdesc(j, b):
                return pltpu.make_async_copy(
                    stage.at[idx_v.at[pl.ds(j * CH, CH)]], rows[b], gsem[b])

            def wdesc(j, b):
                return pltpu.make_async_copy(
                    rows[b], out.at[pl.ds(base + j * CH, CH)], wsem[b])

            for b in range(NB):
                gdesc(b, b).start()

            def outer(jo, _):
                j0 = jo * NB
                for b in range(NB):
                    gdesc(j0 + b, b).wait()
                    wdesc(j0 + b, b).start()
                for b in range(NB):
                    wdesc(j0 + b, b).wait()
                    gdesc(j0 + NB + b, b).start()
                return 0

            lax.fori_loop(0, n_outer - 1, outer, 0)
            j0 = (n_outer - 1) * NB
            for b in range(NB):
                gdesc(j0 + b, b).wait()
                wdesc(j0 + b, b).start()
            for b in range(NB):
                wdesc(j0 + b, b).wait()

        for i in range(ns):
            if tables[i] is not tables[i - 1] or i == 0:
                load_stage(tabs[i])

            def scoped(rows, gsem, wsem, i=i):
                run(tabs[i], ihs[i], outs[i], rows, gsem, wsem)
            pl.run_scoped(
                scoped,
                rows=[pltpu.VMEM((CH, Ws[i]), dts[i]) for _ in range(NB)],
                gsem=[pltpu.SemaphoreType.DMA for _ in range(NB)],
                wsem=[pltpu.SemaphoreType.DMA for _ in range(NB)],
            )
            if i + 1 < ns and tables[i + 1] is not tables[i]:
                plsc.subcore_barrier()

    return k(*tables, *idxs)


def _sc_scatter(vals, idx, n_out, zeros, CH, NB=5):
    E = idx.shape[0]
    W = vals[0].shape[1]
    two = len(vals) == 2
    per_s = (E // 16) if two else (E // 32)
    n_ch = per_s // CH
    assert per_s % CH == 0 and n_ch % NB == 0 and CH % 8 == 0
    n_outer = n_ch // NB
    CZ = 80
    n_zc = n_out // CZ
    n_j = -(-n_zc // 16)
    assert n_out % CZ == 0
    mesh = plsc.VectorSubcoreMesh(core_axis_name="c", subcore_axis_name="s")

    @functools.partial(
        pl.kernel,
        out_type=jax.ShapeDtypeStruct((2 * n_out, W), F32),
        mesh=mesh,
        scratch_types=[
            pltpu.VMEM_SHARED((n_out, W), F32),
            [pltpu.VMEM((CH,), jnp.int32) for _ in range(NB)],
            [pltpu.VMEM((CH, W), F32) for _ in range(NB)],
            pltpu.VMEM((80, W), F32),
            [pltpu.SemaphoreType.DMA for _ in range(NB)],
            [pltpu.SemaphoreType.DMA for _ in range(NB)],
            [pltpu.SemaphoreType.DMA for _ in range(NB)],
            pltpu.SemaphoreType.DMA,
        ],
    )
    def k(*refs):
        if two:
            v0h, v1h, ih, zh, out = refs[:5]
            rest = refs[5:]
        else:
            v0h, ih, zh, out = refs[:4]
            v1h = v0h
            rest = refs[4:]
        acc, idxc, valc, dump, isem, vsem, ssem, zsem = rest
        c = lax.axis_index("c")
        s = lax.axis_index("s")
        pltpu.async_copy(zh.at[pl.ds(0, CZ)], dump, zsem).wait()
        for j in range(n_j):
            cid = s + 16 * j

            @pl.when(cid < n_zc)
            def _():
                pltpu.sync_copy(dump, acc.at[pl.ds(cid * CZ, CZ)])
        plsc.subcore_barrier()

        def run(vh, base):
            def idesc(j, b):
                return pltpu.make_async_copy(
                    ih.at[pl.ds(base + j * CH, CH)], idxc[b], isem[b])

            def vdesc(j, b):
                return pltpu.make_async_copy(
                    vh.at[pl.ds(base + j * CH, CH)], valc[b], vsem[b])

            def sstart(b):
                pltpu.async_copy(valc[b], acc.at[idxc[b]], ssem[b], add=True)

            def swait(b):
                pltpu.make_async_copy(valc[b], acc.at[idxc[b]], ssem[b]).wait()

            for b in range(NB):
                idesc(b, b).start()
                vdesc(b, b).start()

            def outer(jo, _):
                j0 = jo * NB
                for b in range(NB):
                    idesc(j0 + b, b).wait()
                    vdesc(j0 + b, b).wait()
                    sstart(b)
                for b in range(NB):
                    swait(b)
                    idesc(j0 + NB + b, b).start()
                    vdesc(j0 + NB + b, b).start()
                return 0

            lax.fori_loop(0, n_outer - 1, outer, 0)
            for b in range(NB):
                idesc((n_outer - 1) * NB + b, b).wait()
                vdesc((n_outer - 1) * NB + b, b).wait()
                sstart(b)
            for b in range(NB):
                swait(b)

        if two:
            @pl.when(c == 0)
            def _():
                run(v0h, s * per_s)

            @pl.when(c == 1)
            def _():
                run(v1h, s * per_s)
        else:
            run(v0h, (c * 16 + s) * per_s)

        plsc.subcore_barrier()
        for j in range(n_j):
            cid = s + 16 * j

            @pl.when(cid < n_zc)
            def _():
                pltpu.sync_copy(acc.at[pl.ds(cid * CZ, CZ)], dump)
                pltpu.sync_copy(dump, out.at[pl.ds(c * n_out + cid * CZ, CZ)])

    if two:
        return k(vals[0], vals[1], idx, zeros)
    return k(vals[0], idx, zeros)


def _silu(x):
    return x * jax.nn.sigmoid(x)


def _tc_edge_gm(xd, xs, ea, p, B=2000):
    E, D = ea.shape
    nb = E // B

    def body(xd_r, xs_r, ea_r, wg1, bg1, wg2, bg2, wa1, ba1, wa2, ba2,
             g_o, m_o, st_o):
        i = pl.program_id(0)
        xd_ = xd_r[...]
        xs_ = xs_r[...]
        ea_ = ea_r[...]

        def mlp(w1, b1, w2, b2):
            z = (jnp.dot(xd_, w1[0:D], preferred_element_type=F32)
                 + jnp.dot(xs_, w1[D:2 * D], preferred_element_type=F32)
                 + jnp.dot(ea_, w1[2 * D:3 * D], preferred_element_type=F32)
                 + b1[...])
            return jnp.dot(_silu(z), w2[...], preferred_element_type=F32) + b2[...]

        g = mlp(wg1, bg1, wg2, bg2)
        m = mlp(wa1, ba1, wa2, ba2)
        g_o[...] = g.astype(jnp.bfloat16)
        m_o[...] = m.astype(jnp.bfloat16)
        s0 = jnp.sum(g, axis=0, keepdims=True)
        s1 = jnp.sum(g * g, axis=0, keepdims=True)
        upd = jnp.concatenate([s0, s1, jnp.zeros((6, D), F32)], axis=0)
        st_o[...] = jnp.where(i == 0, upd, st_o[...] + upd)

    wspec = lambda shp: pl.BlockSpec(shp, lambda i: (0,) * len(shp))
    espec = pl.BlockSpec((B, D), lambda i: (i, 0))
    return pl.pallas_call(
        body,
        grid=(nb,),
        in_specs=[espec, espec, espec,
                  wspec((3 * D, D)), wspec((1, D)), wspec((D, D)), wspec((1, D)),
                  wspec((3 * D, D)), wspec((1, D)), wspec((D, D)), wspec((1, D))],
        out_specs=[espec, espec, wspec((8, D))],
        out_shape=[jax.ShapeDtypeStruct((E, D), jnp.bfloat16),
                   jax.ShapeDtypeStruct((E, D), jnp.bfloat16),
                   jax.ShapeDtypeStruct((8, D), F32)],
    )(xd, xs, ea,
      p['cart_Wg1'], p['cart_bg1'].reshape(1, D), p['cart_Wg2'],
      p['cart_bg2'].reshape(1, D),
      p['cart_Wa1'], p['cart_ba1'].reshape(1, D), p['cart_Wa2'],
      p['cart_ba2'].reshape(1, D))


def _tc_edge_sig(g, m, st0, st1, E_tot, bng, bnb, B=2000):
    E, D = g.shape
    nb = E // B

    def body(g_r, m_r, st0_r, st1_r, bng_r, bnb_r, sm_o, sg_o):
        st_r = st0_r[...] + st1_r[...]
        mean = st_r[0:1] * (1.0 / E_tot)
        var = st_r[1:2] * (1.0 / E_tot) - mean * mean
        scale = bng_r[...] * lax.rsqrt(var + 1e-5)
        shift = bnb_r[...] - mean * scale
        sig = jax.nn.sigmoid(g_r[...].astype(F32) * scale + shift)
        sm_o[...] = sig * m_r[...].astype(F32)
        sg_o[...] = sig

    wspec = lambda shp: pl.BlockSpec(shp, lambda i: (0,) * len(shp))
    espec = pl.BlockSpec((B, D), lambda i: (i, 0))
    return pl.pallas_call(
        body,
        grid=(nb,),
        in_specs=[espec, espec, wspec((8, D)), wspec((8, D)), wspec((1, D)),
                  wspec((1, D))],
        out_specs=[espec, espec],
        out_shape=[jax.ShapeDtypeStruct((E, D), F32),
                   jax.ShapeDtypeStruct((E, D), F32)],
    )(g, m, st0, st1, bng.reshape(1, D), bnb.reshape(1, D))


def _graph_norm_in(xc, oh, cnt, w, b, ms):
    dn = (((0,), (0,)), ((), ()))
    sums = lax.dot_general(oh, xc, dn, preferred_element_type=F32)
    meang = sums / cnt
    mpn = jnp.dot(oh, meang, preferred_element_type=F32)
    out1 = xc - mpn * ms
    vsum = lax.dot_general(oh, out1 * out1, dn, preferred_element_type=F32) / cnt
    vpn = jnp.dot(oh, vsum, preferred_element_type=F32)
    return out1 * lax.rsqrt(vpn + 1e-5) * w + b


def _onehot_cnt(b2, NG, Nn):
    gi = lax.broadcasted_iota(jnp.int32, (1, NG), 1)
    oh = (b2 == gi).astype(F32)
    ones = jnp.ones((Nn, 1), F32)
    dn = (((0,), (0,)), ((), ()))
    cnt = lax.dot_general(oh, ones, dn, preferred_element_type=F32) + 1e-6
    return oh, cnt


def _tc_addn(arrs, B=2000):
    N, D = arrs[0].shape
    n = len(arrs)
    nb = N // B

    def body(*refs):
        o = refs[-1]
        acc = refs[0][...]
        for r in refs[1:-1]:
            acc = acc + r[...]
        o[...] = acc

    espec = pl.BlockSpec((B, D), lambda i: (i, 0))
    return pl.pallas_call(
        body,
        grid=(nb,),
        in_specs=[espec] * n,
        out_specs=espec,
        out_shape=jax.ShapeDtypeStruct((N, D), F32),
    )(*arrs)


def _tc_node_c1(x, num, den, batch2, p, NG):
    N, D = x.shape

    def body(x_r, num_r, den_r, b2_r, bng_r, bnb_r, gw_r, gb_r, gms_r, h_o):
        r = num_r[...] / (den_r[...] + 1e-6)
        mean = jnp.mean(r, axis=0, keepdims=True)
        var = jnp.mean((r - mean) ** 2, axis=0, keepdims=True)
        bnr = bng_r[...] * (r - mean) * lax.rsqrt(var + 1e-5) + bnb_r[...]
        xc = x_r[...] + _silu(bnr)
        oh, cnt = _onehot_cnt(b2_r[...], NG, N)
        gn = _graph_norm_in(xc, oh, cnt, gw_r[...], gb_r[...], gms_r[...])
        h_o[...] = x_r[...] + gn

    full = lambda shp: pl.BlockSpec(shp, lambda: (0,) * len(shp))
    return pl.pallas_call(
        body,
        in_specs=[full((N, D)), full((N, D)), full((N, D)), full((N, 1)),
                  full((1, D)), full((1, D)), full((1, D)), full((1, D)),
                  full((1, D))],
        out_specs=full((N, D)),
        out_shape=jax.ShapeDtypeStruct((N, D), F32),
    )(x, num, den, batch2,
      p['cart_bnx_g'].reshape(1, D), p['cart_bnx_b'].reshape(1, D),
      p['gnc_w'].reshape(1, D), p['gnc_b'].reshape(1, D),
      p['gnc_ms'].reshape(1, D))


_MASK_HI = -65536
_RND = 32768


def _pack2(hi_f32, lo_f32):
    hb = lax.bitcast_convert_type(hi_f32, jnp.int32) + _RND
    lb = lax.bitcast_convert_type(lo_f32, jnp.int32) + _RND
    return jnp.bitwise_or(jnp.bitwise_and(hb, _MASK_HI),
                          lax.shift_right_logical(lb, 16))


def _unpack_hi(p):
    return lax.bitcast_convert_type(jnp.bitwise_and(p, _MASK_HI), F32)


def _unpack_lo(p):
    return lax.bitcast_convert_type(jnp.left_shift(p, 16), F32)


def _tc_node_c2(h, p, B=2000):
    N, D = h.shape
    nb = N // B

    def body(h_r, wq, bq, wk, bk, wv, bv, ws, bs, qk_o, v_o, kv_o, xr_o):
        h_ = h_r[...]
        q = jnp.dot(h_, wq[...], preferred_element_type=F32) + bq[...]
        k = jnp.dot(h_, wk[...], preferred_element_type=F32) + bk[...]
        v = jnp.dot(h_, wv[...], preferred_element_type=F32) + bv[...]
        qk_o[...] = _pack2(q, k)
        v_o[...] = lax.bitcast_convert_type(v, jnp.int32)
        kv_o[...] = _pack2(k, v)
        xr_o[...] = jnp.dot(h_, ws[...], preferred_element_type=F32) + bs[...]

    wspec = lambda shp: pl.BlockSpec(shp, lambda i: (0,) * len(shp))
    return pl.pallas_call(
        body,
        grid=(nb,),
        in_specs=[pl.BlockSpec((B, D), lambda i: (i, 0)),
                  wspec((D, D)), wspec((1, D)), wspec((D, D)), wspec((1, D)),
                  wspec((D, D)), wspec((1, D)), wspec((D, D)), wspec((1, D))],
        out_specs=[pl.BlockSpec((B, D), lambda i: (i, 0)),
                   pl.BlockSpec((B, D), lambda i: (i, 0)),
                   pl.BlockSpec((B, D), lambda i: (i, 0)),
                   pl.BlockSpec((B, D), lambda i: (i, 0))],
        out_shape=[jax.ShapeDtypeStruct((N, D), jnp.int32),
                   jax.ShapeDtypeStruct((N, D), jnp.int32),
                   jax.ShapeDtypeStruct((N, D), jnp.int32),
                   jax.ShapeDtypeStruct((N, D), F32)],
    )(h, p['mat_Wq'], p['mat_bq'].reshape(1, D), p['mat_Wk'],
      p['mat_bk'].reshape(1, D), p['mat_Wv'], p['mat_bv'].reshape(1, D),
      p['mat_Wskip'], p['mat_bskip'].reshape(1, D))


def _tc_edge_msg(qkd, vd, kvs, ea, p, B=2000):
    E = qkd.shape[0]
    D = ea.shape[1]
    nb = E // B
    s3 = float(1.0 / np.sqrt(3.0 * D))

    def body(qk_r, vd_r, kv_r, ea_r, we, wmu, bmu, wml, bml, lag, lab,
             lmg, lmb, msg_o):
        qk_ = qk_r[...]
        kv_ = kv_r[...]
        e = jnp.dot(ea_r[...], we[...], preferred_element_type=F32)
        q_i = _unpack_hi(qk_)
        k_i = _unpack_lo(qk_)
        v_i = lax.bitcast_convert_type(vd_r[...], F32)
        k_j = _unpack_hi(kv_)
        v_j = _unpack_lo(kv_)
        a1 = q_i * k_i * s3
        a2 = q_i * k_j * s3
        a3 = q_i * e * s3
        rm = jnp.sum(a1 + a2 + a3, axis=-1, keepdims=True) * (1.0 / (3 * D))
        d1 = a1 - rm
        d2 = a2 - rm
        d3 = a3 - rm
        vv = (jnp.sum(d1 * d1 + d2 * d2 + d3 * d3, axis=-1, keepdims=True)
              * (1.0 / (3 * D)))
        inv = lax.rsqrt(vv + 1e-5)
        lag_ = lag[...]
        lab_ = lab[...]
        gt1 = jax.nn.sigmoid(d1 * inv * lag_[:, 0:D] + lab_[:, 0:D])
        gt2 = jax.nn.sigmoid(d2 * inv * lag_[:, D:2 * D] + lab_[:, D:2 * D])
        gt3 = jax.nn.sigmoid(d3 * inv * lag_[:, 2 * D:3 * D]
                             + lab_[:, 2 * D:3 * D])
        mm = (jnp.dot(v_i, wmu[0:D], preferred_element_type=F32)
              + jnp.dot(v_j, wmu[D:2 * D], preferred_element_type=F32)
              + jnp.dot(e, wmu[2 * D:3 * D], preferred_element_type=F32)
              + bmu[...])
        gated = mm * jnp.concatenate([gt1, gt2, gt3], axis=-1)
        msg2 = jnp.dot(gated, wml[...], preferred_element_type=F32) + bml[...]
        m2 = jnp.mean(msg2, axis=-1, keepdims=True)
        v2 = jnp.mean((msg2 - m2) ** 2, axis=-1, keepdims=True)
        msg_o[...] = lmg[...] * (msg2 - m2) * lax.rsqrt(v2 + 1e-5) + lmb[...]

    wspec = lambda shp: pl.BlockSpec(shp, lambda i: (0,) * len(shp))
    return pl.pallas_call(
        body,
        grid=(nb,),
        in_specs=[pl.BlockSpec((B, D), lambda i: (i, 0)),
                  pl.BlockSpec((B, D), lambda i: (i, 0)),
                  pl.BlockSpec((B, D), lambda i: (i, 0)),
                  pl.BlockSpec((B, D), lambda i: (i, 0)),
                  wspec((D, D)), wspec((3 * D, 3 * D)), wspec((1, 3 * D)),
                  wspec((3 * D, D)), wspec((1, D)),
                  wspec((1, 3 * D)), wspec((1, 3 * D)),
                  wspec((1, D)), wspec((1, D))],
        out_specs=pl.BlockSpec((B, D), lambda i: (i, 0)),
        out_shape=jax.ShapeDtypeStruct((E, D), F32),
    )(qkd, vd, kvs, ea, p['mat_We'], p['mat_Wmu'],
      p['mat_bmu'].reshape(1, 3 * D),
      p['mat_Wml'], p['mat_bml'].reshape(1, D),
      p['mat_lna_g'].reshape(1, 3 * D), p['mat_lna_b'].reshape(1, 3 * D),
      p['mat_lnm_g'].reshape(1, D), p['mat_lnm_b'].reshape(1, D))


def _tc_node_f(out, h, xr, batch2, p, NG):
    N, D = h.shape

    def body(o_r, h_r, xr_r, b2_r, wb, gw, gb, gms, f_o):
        out = o_r[...]
        xr_ = xr_r[...]
        wb_ = wb[...]
        bl = (jnp.dot(out, wb_[0:D], preferred_element_type=F32)
              + jnp.dot(xr_, wb_[D:2 * D], preferred_element_type=F32)
              + jnp.dot(out - xr_, wb_[2 * D:3 * D],
                        preferred_element_type=F32))
        beta = jax.nn.sigmoid(bl)
        hm = beta * xr_ + (1.0 - beta) * out
        oh, cnt = _onehot_cnt(b2_r[...], NG, N)
        gn = _graph_norm_in(hm, oh, cnt, gw[...], gb[...], gms[...])
        f_o[...] = h_r[...] + gn

    full = lambda shp: pl.BlockSpec(shp, lambda: (0,) * len(shp))
    return pl.pallas_call(
        body,
        in_specs=[full((N, D)), full((N, D)), full((N, D)),
                  full((N, 1)), full((3 * D, 1)),
                  full((1, D)), full((1, D)), full((1, D))],
        out_specs=full((N, D)),
        out_shape=jax.ShapeDtypeStruct((N, D), F32),
    )(out, h, xr, batch2, p['mat_Wbeta'],
      p['gnm_w'].reshape(1, D), p['gnm_b'].reshape(1, D),
      p['gnm_ms'].reshape(1, D))


def kernel(x, edge_index, edge_attr, batch, params):
    p = params
    N, D = x.shape
    E = edge_index.shape[1]
    NG = 64
    src = edge_index[0]
    dst = edge_index[1]
    batch2 = batch.reshape(N, 1)
    zeros = jnp.zeros((N, D), F32)

    E0 = (E * 3) // 5
    sl0 = slice(0, E0)
    sl1 = slice(E0, E)
    dst0, dst1 = dst[sl0], dst[sl1]
    src0, src1 = src[sl0], src[sl1]
    ea0, ea1 = edge_attr[sl0], edge_attr[sl1]

    xd0, xs0 = _sc_gather_multi([x, x], [dst0, src0])
    xd1, xs1 = _sc_gather_multi([x, x], [dst1, src1])
    g0, m0, st0 = _tc_edge_gm(xd0, xs0, ea0, p)
    g1, m1, st1 = _tc_edge_gm(xd1, xs1, ea1, p)
    sm0, sig0 = _tc_edge_sig(g0, m0, st0, st1, E, p['cart_bne_g'],
                             p['cart_bne_b'])
    sm1, sig1 = _tc_edge_sig(g1, m1, st0, st1, E, p['cart_bne_g'],
                             p['cart_bne_b'])
    nd0 = _sc_scatter([sm0, sig0], dst0, N, zeros, CH=40)
    nd1 = _sc_scatter([sm1, sig1], dst1, N, zeros, CH=40)
    num = _tc_addn([nd0[0:N], nd1[0:N]])
    den = _tc_addn([nd0[N:2 * N], nd1[N:2 * N]])
    h = _tc_node_c1(x, num, den, batch2, p, NG)

    qk, vv, kv, xr = _tc_node_c2(h, p)
    qkd0, vd0, kvs0 = _sc_gather_multi([qk, vv, kv], [dst0, dst0, src0])
    qkd1, vd1, kvs1 = _sc_gather_multi([qk, vv, kv], [dst1, dst1, src1])
    msg0 = _tc_edge_msg(qkd0, vd0, kvs0, ea0, p)
    msg1 = _tc_edge_msg(qkd1, vd1, kvs1, ea1, p)
    op0 = _sc_scatter([msg0], dst0, N, zeros, CH=40)
    op1 = _sc_scatter([msg1], dst1, N, zeros, CH=40)
    out = _tc_addn([op0[0:N], op0[N:2 * N], op1[0:N], op1[N:2 * N]])
    hf = _tc_node_f(out, h, xr, batch2, p, NG)
    return hf


# speedup vs baseline: 1.0535x; 1.0535x over previous
import functools

import jax
import jax.numpy as jnp
import numpy as np
from jax import lax
from jax.experimental import pallas as pl
from jax.experimental.pallas import tpu as pltpu
from jax.experimental.pallas import tpu_sc as plsc

F32 = jnp.float32


def _sc_gather_multi(tables, idxs, CH=40, NB=5):
    E = idxs[0].shape[0]
    ns = len(tables)
    Ws = [t.shape[1] for t in tables]
    dts = [t.dtype for t in tables]
    Nt = tables[0].shape[0]
    assert all(t.shape == (Nt, Ws[0]) and t.dtype == dts[0] for t in tables)
    per_w = E // 32
    n_ch = per_w // CH
    assert per_w % CH == 0 and n_ch % NB == 0 and CH % 8 == 0
    n_outer = n_ch // NB
    CZ = 80
    n_zc = Nt // CZ
    n_j = -(-n_zc // 16)
    assert Nt % CZ == 0
    mesh = plsc.VectorSubcoreMesh(core_axis_name="c", subcore_axis_name="s")

    @functools.partial(
        pl.kernel,
        out_type=tuple(
            jax.ShapeDtypeStruct((E, Ws[i]), dts[i]) for i in range(ns)),
        mesh=mesh,
        scratch_types=[pltpu.VMEM((per_w,), jnp.int32),
                       pltpu.VMEM_SHARED((Nt, Ws[0]), dts[0])],
    )
    def k(*refs):
        tabs = refs[:ns]
        ihs = refs[ns:2 * ns]
        outs = refs[2 * ns:3 * ns]
        idx_v = refs[3 * ns]
        stage = refs[3 * ns + 1]
        c = lax.axis_index("c")
        s = lax.axis_index("s")
        base = (s * 2 + c) * per_w

        def load_stage(tab):
            for j in range(n_j):
                cid = s + 16 * j

                @pl.when(cid < n_zc)
                def _():
                    pltpu.sync_copy(tab.at[pl.ds(cid * CZ, CZ)],
                                    stage.at[pl.ds(cid * CZ, CZ)])
            plsc.subcore_barrier()

        def run(tab, ih, out, rows, gsem, wsem):
            pltpu.sync_copy(ih.at[pl.ds(base, per_w)], idx_v)

            def gdesc(j, b):
                return pltpu.make_async_copy(
                    stage.at[idx_v.at[pl.ds(j * CH, CH)]], rows[b], gsem[b])

            def wdesc(j, b):
                return pltpu.make_async_copy(
                    rows[b], out.at[pl.ds(base + j * CH, CH)], wsem[b])

            for b in range(NB):
                gdesc(b, b).start()

            def outer(jo, _):
                j0 = jo * NB
                for b in range(NB):
                    gdesc(j0 + b, b).wait()
                    wdesc(j0 + b, b).start()
                for b in range(NB):
                    wdesc(j0 + b, b).wait()
                    gdesc(j0 + NB + b, b).start()
                return 0

            lax.fori_loop(0, n_outer - 1, outer, 0)
            j0 = (n_outer - 1) * NB
            for b in range(NB):
                gdesc(j0 + b, b).wait()
                wdesc(j0 + b, b).start()
            for b in range(NB):
                wdesc(j0 + b, b).wait()

        for i in range(ns):
            if tables[i] is not tables[i - 1] or i == 0:
                load_stage(tabs[i])

            def scoped(rows, gsem, wsem, i=i):
                run(tabs[i], ihs[i], outs[i], rows, gsem, wsem)
            pl.run_scoped(
                scoped,
                rows=[pltpu.VMEM((CH, Ws[i]), dts[i]) for _ in range(NB)],
                gsem=[pltpu.SemaphoreType.DMA for _ in range(NB)],
                wsem=[pltpu.SemaphoreType.DMA for _ in range(NB)],
            )
            if i + 1 < ns and tables[i + 1] is not tables[i]:
                plsc.subcore_barrier()

    return k(*tables, *idxs)


def _sc_scatter(vals, idx, n_out, zeros, CH, NB=5):
    E = idx.shape[0]
    W = vals[0].shape[1]
    two = len(vals) == 2
    per_s = (E // 16) if two else (E // 32)
    n_ch = per_s // CH
    assert per_s % CH == 0 and n_ch % NB == 0 and CH % 8 == 0
    n_outer = n_ch // NB
    CZ = 80
    n_zc = n_out // CZ
    n_j = -(-n_zc // 16)
    assert n_out % CZ == 0
    mesh = plsc.VectorSubcoreMesh(core_axis_name="c", subcore_axis_name="s")

    @functools.partial(
        pl.kernel,
        out_type=jax.ShapeDtypeStruct((2 * n_out, W), F32),
        mesh=mesh,
        scratch_types=[
            pltpu.VMEM_SHARED((n_out, W), F32),
            [pltpu.VMEM((CH,), jnp.int32) for _ in range(NB)],
            [pltpu.VMEM((CH, W), F32) for _ in range(NB)],
            pltpu.VMEM((80, W), F32),
            [pltpu.SemaphoreType.DMA for _ in range(NB)],
            [pltpu.SemaphoreType.DMA for _ in range(NB)],
            [pltpu.SemaphoreType.DMA for _ in range(NB)],
            pltpu.SemaphoreType.DMA,
        ],
    )
    def k(*refs):
        if two:
            v0h, v1h, ih, zh, out = refs[:5]
            rest = refs[5:]
        else:
            v0h, ih, zh, out = refs[:4]
            v1h = v0h
            rest = refs[4:]
        acc, idxc, valc, dump, isem, vsem, ssem, zsem = rest
        c = lax.axis_index("c")
        s = lax.axis_index("s")
        pltpu.async_copy(zh.at[pl.ds(0, CZ)], dump, zsem).wait()
        for j in range(n_j):
            cid = s + 16 * j

            @pl.when(cid < n_zc)
            def _():
                pltpu.sync_copy(dump, acc.at[pl.ds(cid * CZ, CZ)])
        plsc.subcore_barrier()

        def run(vh, base):
            def idesc(j, b):
                return pltpu.make_async_copy(
                    ih.at[pl.ds(base + j * CH, CH)], idxc[b], isem[b])

            def vdesc(j, b):
                return pltpu.make_async_copy(
                    vh.at[pl.ds(base + j * CH, CH)], valc[b], vsem[b])

            def sstart(b):
                pltpu.async_copy(valc[b], acc.at[idxc[b]], ssem[b], add=True)

            def swait(b):
                pltpu.make_async_copy(valc[b], acc.at[idxc[b]], ssem[b]).wait()

            for b in range(NB):
                idesc(b, b).start()
                vdesc(b, b).start()

            def outer(jo, _):
                j0 = jo * NB
                for b in range(NB):
                    idesc(j0 + b, b).wait()
                    vdesc(j0 + b, b).wait()
                    sstart(b)
                for b in range(NB):
                    swait(b)
                    idesc(j0 + NB + b, b).start()
                    vdesc(j0 + NB + b, b).start()
                return 0

            lax.fori_loop(0, n_outer - 1, outer, 0)
            for b in range(NB):
                idesc((n_outer - 1) * NB + b, b).wait()
                vdesc((n_outer - 1) * NB + b, b).wait()
                sstart(b)
            for b in range(NB):
                swait(b)

        if two:
            @pl.when(c == 0)
            def _():
                run(v0h, s * per_s)

            @pl.when(c == 1)
            def _():
                run(v1h, s * per_s)
        else:
            run(v0h, (c * 16 + s) * per_s)

        plsc.subcore_barrier()
        for j in range(n_j):
            cid = s + 16 * j

            @pl.when(cid < n_zc)
            def _():
                pltpu.sync_copy(acc.at[pl.ds(cid * CZ, CZ)], dump)
                pltpu.sync_copy(dump, out.at[pl.ds(c * n_out + cid * CZ, CZ)])

    if two:
        return k(vals[0], vals[1], idx, zeros)
    return k(vals[0], idx, zeros)


def _silu(x):
    return x * jax.nn.sigmoid(x)


def _tc_edge_gm(xd, xs, ea, p, B=4000):
    E, D = ea.shape
    nb = E // B

    def body(xd_r, xs_r, ea_r, wg1, bg1, wg2, bg2, wa1, ba1, wa2, ba2,
             g_o, m_o, st_o):
        i = pl.program_id(0)
        xd_ = xd_r[...]
        xs_ = xs_r[...]
        ea_ = ea_r[...]

        def mlp(w1, b1, w2, b2):
            z = (jnp.dot(xd_, w1[0:D], preferred_element_type=F32)
                 + jnp.dot(xs_, w1[D:2 * D], preferred_element_type=F32)
                 + jnp.dot(ea_, w1[2 * D:3 * D], preferred_element_type=F32)
                 + b1[...])
            return jnp.dot(_silu(z), w2[...], preferred_element_type=F32) + b2[...]

        g = mlp(wg1, bg1, wg2, bg2)
        m = mlp(wa1, ba1, wa2, ba2)
        g_o[...] = g.astype(jnp.bfloat16)
        m_o[...] = m.astype(jnp.bfloat16)
        s0 = jnp.sum(g, axis=0, keepdims=True)
        s1 = jnp.sum(g * g, axis=0, keepdims=True)
        upd = jnp.concatenate([s0, s1, jnp.zeros((6, D), F32)], axis=0)
        st_o[...] = jnp.where(i == 0, upd, st_o[...] + upd)

    wspec = lambda shp: pl.BlockSpec(shp, lambda i: (0,) * len(shp))
    espec = pl.BlockSpec((B, D), lambda i: (i, 0))
    return pl.pallas_call(
        body,
        grid=(nb,),
        in_specs=[espec, espec, espec,
                  wspec((3 * D, D)), wspec((1, D)), wspec((D, D)), wspec((1, D)),
                  wspec((3 * D, D)), wspec((1, D)), wspec((D, D)), wspec((1, D))],
        out_specs=[espec, espec, wspec((8, D))],
        out_shape=[jax.ShapeDtypeStruct((E, D), jnp.bfloat16),
                   jax.ShapeDtypeStruct((E, D), jnp.bfloat16),
                   jax.ShapeDtypeStruct((8, D), F32)],
    )(xd, xs, ea,
      p['cart_Wg1'], p['cart_bg1'].reshape(1, D), p['cart_Wg2'],
      p['cart_bg2'].reshape(1, D),
      p['cart_Wa1'], p['cart_ba1'].reshape(1, D), p['cart_Wa2'],
      p['cart_ba2'].reshape(1, D))


def _tc_edge_sig(g, m, st, bng, bnb, B=4000):
    E, D = g.shape
    nb = E // B

    def body(g_r, m_r, st_r, bng_r, bnb_r, sm_o, sg_o):
        mean = st_r[0:1] * (1.0 / E)
        var = st_r[1:2] * (1.0 / E) - mean * mean
        scale = bng_r[...] * lax.rsqrt(var + 1e-5)
        shift = bnb_r[...] - mean * scale
        sig = jax.nn.sigmoid(g_r[...].astype(F32) * scale + shift)
        sm_o[...] = sig * m_r[...].astype(F32)
        sg_o[...] = sig

    wspec = lambda shp: pl.BlockSpec(shp, lambda i: (0,) * len(shp))
    espec = pl.BlockSpec((B, D), lambda i: (i, 0))
    return pl.pallas_call(
        body,
        grid=(nb,),
        in_specs=[espec, espec, wspec((8, D)), wspec((1, D)), wspec((1, D))],
        out_specs=[espec, espec],
        out_shape=[jax.ShapeDtypeStruct((E, D), F32),
                   jax.ShapeDtypeStruct((E, D), F32)],
    )(g, m, st, bng.reshape(1, D), bnb.reshape(1, D))


def _graph_norm_in(xc, oh, cnt, w, b, ms):
    dn = (((0,), (0,)), ((), ()))
    sums = lax.dot_general(oh, xc, dn, preferred_element_type=F32)
    meang = sums / cnt
    mpn = jnp.dot(oh, meang, preferred_element_type=F32)
    out1 = xc - mpn * ms
    vsum = lax.dot_general(oh, out1 * out1, dn, preferred_element_type=F32) / cnt
    vpn = jnp.dot(oh, vsum, preferred_element_type=F32)
    return out1 * lax.rsqrt(vpn + 1e-5) * w + b


def _onehot_cnt(b2, NG, Nn):
    gi = lax.broadcasted_iota(jnp.int32, (1, NG), 1)
    oh = (b2 == gi).astype(F32)
    ones = jnp.ones((Nn, 1), F32)
    dn = (((0,), (0,)), ((), ()))
    cnt = lax.dot_general(oh, ones, dn, preferred_element_type=F32) + 1e-6
    return oh, cnt


def _tc_node_c1(x, num, den, batch2, p, NG):
    N, D = x.shape

    def body(x_r, num_r, den_r, b2_r, bng_r, bnb_r, gw_r, gb_r, gms_r, h_o):
        r = num_r[...] / (den_r[...] + 1e-6)
        mean = jnp.mean(r, axis=0, keepdims=True)
        var = jnp.mean((r - mean) ** 2, axis=0, keepdims=True)
        bnr = bng_r[...] * (r - mean) * lax.rsqrt(var + 1e-5) + bnb_r[...]
        xc = x_r[...] + _silu(bnr)
        oh, cnt = _onehot_cnt(b2_r[...], NG, N)
        gn = _graph_norm_in(xc, oh, cnt, gw_r[...], gb_r[...], gms_r[...])
        h_o[...] = x_r[...] + gn

    full = lambda shp: pl.BlockSpec(shp, lambda: (0,) * len(shp))
    return pl.pallas_call(
        body,
        in_specs=[full((N, D)), full((N, D)), full((N, D)), full((N, 1)),
                  full((1, D)), full((1, D)), full((1, D)), full((1, D)),
                  full((1, D))],
        out_specs=full((N, D)),
        out_shape=jax.ShapeDtypeStruct((N, D), F32),
    )(x, num, den, batch2,
      p['cart_bnx_g'].reshape(1, D), p['cart_bnx_b'].reshape(1, D),
      p['gnc_w'].reshape(1, D), p['gnc_b'].reshape(1, D),
      p['gnc_ms'].reshape(1, D))


_MASK_HI = -65536
_RND = 32768


def _pack2(hi_f32, lo_f32):
    hb = lax.bitcast_convert_type(hi_f32, jnp.int32) + _RND
    lb = lax.bitcast_convert_type(lo_f32, jnp.int32) + _RND
    return jnp.bitwise_or(jnp.bitwise_and(hb, _MASK_HI),
                          lax.shift_right_logical(lb, 16))


def _unpack_hi(p):
    return lax.bitcast_convert_type(jnp.bitwise_and(p, _MASK_HI), F32)


def _unpack_lo(p):
    return lax.bitcast_convert_type(jnp.left_shift(p, 16), F32)


def _tc_node_c2(h, p, B=2000):
    N, D = h.shape
    nb = N // B

    def body(h_r, wq, bq, wk, bk, wv, bv, ws, bs, qk_o, v_o, kv_o, xr_o):
        h_ = h_r[...]
        q = jnp.dot(h_, wq[...], preferred_element_type=F32) + bq[...]
        k = jnp.dot(h_, wk[...], preferred_element_type=F32) + bk[...]
        v = jnp.dot(h_, wv[...], preferred_element_type=F32) + bv[...]
        qk_o[...] = _pack2(q, k)
        v_o[...] = lax.bitcast_convert_type(v, jnp.int32)
        kv_o[...] = _pack2(k, v)
        xr_o[...] = jnp.dot(h_, ws[...], preferred_element_type=F32) + bs[...]

    wspec = lambda shp: pl.BlockSpec(shp, lambda i: (0,) * len(shp))
    return pl.pallas_call(
        body,
        grid=(nb,),
        in_specs=[pl.BlockSpec((B, D), lambda i: (i, 0)),
                  wspec((D, D)), wspec((1, D)), wspec((D, D)), wspec((1, D)),
                  wspec((D, D)), wspec((1, D)), wspec((D, D)), wspec((1, D))],
        out_specs=[pl.BlockSpec((B, D), lambda i: (i, 0)),
                   pl.BlockSpec((B, D), lambda i: (i, 0)),
                   pl.BlockSpec((B, D), lambda i: (i, 0)),
                   pl.BlockSpec((B, D), lambda i: (i, 0))],
        out_shape=[jax.ShapeDtypeStruct((N, D), jnp.int32),
                   jax.ShapeDtypeStruct((N, D), jnp.int32),
                   jax.ShapeDtypeStruct((N, D), jnp.int32),
                   jax.ShapeDtypeStruct((N, D), F32)],
    )(h, p['mat_Wq'], p['mat_bq'].reshape(1, D), p['mat_Wk'],
      p['mat_bk'].reshape(1, D), p['mat_Wv'], p['mat_bv'].reshape(1, D),
      p['mat_Wskip'], p['mat_bskip'].reshape(1, D))


def _tc_edge_msg(qkd, vd, kvs, ea, p, B=4000):
    E = qkd.shape[0]
    D = ea.shape[1]
    nb = E // B
    s3 = float(1.0 / np.sqrt(3.0 * D))

    def body(qk_r, vd_r, kv_r, ea_r, we, wmu, bmu, wml, bml, lag, lab,
             lmg, lmb, msg_o):
        qk_ = qk_r[...]
        kv_ = kv_r[...]
        e = jnp.dot(ea_r[...], we[...], preferred_element_type=F32)
        q_i = _unpack_hi(qk_)
        k_i = _unpack_lo(qk_)
        v_i = lax.bitcast_convert_type(vd_r[...], F32)
        k_j = _unpack_hi(kv_)
        v_j = _unpack_lo(kv_)
        a1 = q_i * k_i * s3
        a2 = q_i * k_j * s3
        a3 = q_i * e * s3
        rm = jnp.sum(a1 + a2 + a3, axis=-1, keepdims=True) * (1.0 / (3 * D))
        d1 = a1 - rm
        d2 = a2 - rm
        d3 = a3 - rm
        vv = (jnp.sum(d1 * d1 + d2 * d2 + d3 * d3, axis=-1, keepdims=True)
              * (1.0 / (3 * D)))
        inv = lax.rsqrt(vv + 1e-5)
        lag_ = lag[...]
        lab_ = lab[...]
        gt1 = jax.nn.sigmoid(d1 * inv * lag_[:, 0:D] + lab_[:, 0:D])
        gt2 = jax.nn.sigmoid(d2 * inv * lag_[:, D:2 * D] + lab_[:, D:2 * D])
        gt3 = jax.nn.sigmoid(d3 * inv * lag_[:, 2 * D:3 * D]
                             + lab_[:, 2 * D:3 * D])
        mm = (jnp.dot(v_i, wmu[0:D], preferred_element_type=F32)
              + jnp.dot(v_j, wmu[D:2 * D], preferred_element_type=F32)
              + jnp.dot(e, wmu[2 * D:3 * D], preferred_element_type=F32)
              + bmu[...])
        gated = mm * jnp.concatenate([gt1, gt2, gt3], axis=-1)
        msg2 = jnp.dot(gated, wml[...], preferred_element_type=F32) + bml[...]
        m2 = jnp.mean(msg2, axis=-1, keepdims=True)
        v2 = jnp.mean((msg2 - m2) ** 2, axis=-1, keepdims=True)
        msg_o[...] = lmg[...] * (msg2 - m2) * lax.rsqrt(v2 + 1e-5) + lmb[...]

    wspec = lambda shp: pl.BlockSpec(shp, lambda i: (0,) * len(shp))
    return pl.pallas_call(
        body,
        grid=(nb,),
        in_specs=[pl.BlockSpec((B, D), lambda i: (i, 0)),
                  pl.BlockSpec((B, D), lambda i: (i, 0)),
                  pl.BlockSpec((B, D), lambda i: (i, 0)),
                  pl.BlockSpec((B, D), lambda i: (i, 0)),
                  wspec((D, D)), wspec((3 * D, 3 * D)), wspec((1, 3 * D)),
                  wspec((3 * D, D)), wspec((1, D)),
                  wspec((1, 3 * D)), wspec((1, 3 * D)),
                  wspec((1, D)), wspec((1, D))],
        out_specs=pl.BlockSpec((B, D), lambda i: (i, 0)),
        out_shape=jax.ShapeDtypeStruct((E, D), F32),
    )(qkd, vd, kvs, ea, p['mat_We'], p['mat_Wmu'],
      p['mat_bmu'].reshape(1, 3 * D),
      p['mat_Wml'], p['mat_bml'].reshape(1, D),
      p['mat_lna_g'].reshape(1, 3 * D), p['mat_lna_b'].reshape(1, 3 * D),
      p['mat_lnm_g'].reshape(1, D), p['mat_lnm_b'].reshape(1, D))


def _tc_node_f(out0, out1, h, xr, batch2, p, NG):
    N, D = h.shape

    def body(o0_r, o1_r, h_r, xr_r, b2_r, wb, gw, gb, gms, f_o):
        out = o0_r[...] + o1_r[...]
        xr_ = xr_r[...]
        wb_ = wb[...]
        bl = (jnp.dot(out, wb_[0:D], preferred_element_type=F32)
              + jnp.dot(xr_, wb_[D:2 * D], preferred_element_type=F32)
              + jnp.dot(out - xr_, wb_[2 * D:3 * D],
                        preferred_element_type=F32))
        beta = jax.nn.sigmoid(bl)
        hm = beta * xr_ + (1.0 - beta) * out
        oh, cnt = _onehot_cnt(b2_r[...], NG, N)
        gn = _graph_norm_in(hm, oh, cnt, gw[...], gb[...], gms[...])
        f_o[...] = h_r[...] + gn

    full = lambda shp: pl.BlockSpec(shp, lambda: (0,) * len(shp))
    return pl.pallas_call(
        body,
        in_specs=[full((N, D)), full((N, D)), full((N, D)), full((N, D)),
                  full((N, 1)), full((3 * D, 1)),
                  full((1, D)), full((1, D)), full((1, D))],
        out_specs=full((N, D)),
        out_shape=jax.ShapeDtypeStruct((N, D), F32),
    )(out0, out1, h, xr, batch2, p['mat_Wbeta'],
      p['gnm_w'].reshape(1, D), p['gnm_b'].reshape(1, D),
      p['gnm_ms'].reshape(1, D))


def kernel(x, edge_index, edge_attr, batch, params):
    p = params
    N, D = x.shape
    E = edge_index.shape[1]
    NG = 64
    src = edge_index[0]
    dst = edge_index[1]
    batch2 = batch.reshape(N, 1)
    zeros = jnp.zeros((N, D), F32)

    xd, xs = _sc_gather_multi([x, x], [dst, src])
    g, m, st = _tc_edge_gm(xd, xs, edge_attr, p)
    sm, sig = _tc_edge_sig(g, m, st, p['cart_bne_g'], p['cart_bne_b'])
    nd = _sc_scatter([sm, sig], dst, N, zeros, CH=40)
    num = nd[0:N]
    den = nd[N:2 * N]
    h = _tc_node_c1(x, num, den, batch2, p, NG)

    qk, vv, kv, xr = _tc_node_c2(h, p)
    qkd, vd, kvs = _sc_gather_multi([qk, vv, kv], [dst, dst, src])
    msg = _tc_edge_msg(qkd, vd, kvs, edge_attr, p)
    op = _sc_scatter([msg], dst, N, zeros, CH=40)
    hf = _tc_node_f(op[0:N], op[N:2 * N], h, xr, batch2, p, NG)
    return hf
